# Initial kernel scaffold; baseline (speedup 1.0000x reference)
import functools

import jax
import jax.numpy as jnp
from jax import lax
from jax.experimental import pallas as pl
from jax.experimental.pallas import tpu as pltpu
from jax.experimental.pallas import tpu_sc as plsc

B_TOTAL = 16384 * 50
D = 64
NC, NS = 2, 16
NW = NC * NS
BPW = B_TOTAL // NW
CHUNK = 128
NCHUNK = BPW // CHUNK

_mesh = plsc.VectorSubcoreMesh(core_axis_name="c", subcore_axis_name="s")


@functools.partial(
    pl.kernel,
    mesh=_mesh,
    out_type=jax.ShapeDtypeStruct((B_TOTAL, D), jnp.float32),
    scratch_types=[
        pltpu.VMEM((NCHUNK, CHUNK), jnp.int32),
        pltpu.VMEM((CHUNK, D), jnp.float32),
        pltpu.SemaphoreType.DMA,
    ],
)
def _emb_lookup(x_hbm, table_hbm, out_hbm, idx_v, rows_v, sem):
    wid = lax.axis_index("s") * NC + lax.axis_index("c")
    base = wid * BPW
    pltpu.sync_copy(x_hbm.at[wid], idx_v)

    def step(g, carry):
        pltpu.async_copy(table_hbm.at[idx_v.at[g]], rows_v, sem).wait()
        pltpu.sync_copy(rows_v, out_hbm.at[pl.ds(base + g * CHUNK, CHUNK)])
        return carry

    lax.fori_loop(0, NCHUNK, step, 0)


def kernel(x, table):
    xf = x.reshape(NW, NCHUNK, CHUNK).astype(jnp.int32)
    out = _emb_lookup(xf, table)
    return out.reshape(x.shape[0], x.shape[1], D)


# speedup vs baseline: 1.6859x; 1.6859x over previous
import functools

import jax
import jax.numpy as jnp
from jax import lax
from jax.experimental import pallas as pl
from jax.experimental.pallas import tpu as pltpu
from jax.experimental.pallas import tpu_sc as plsc

B_TOTAL = 16384 * 50
D = 64
NC, NS = 2, 16
NW = NC * NS
BPW = B_TOTAL // NW
CHUNK = 128
NCHUNK = BPW // CHUNK

_mesh = plsc.VectorSubcoreMesh(core_axis_name="c", subcore_axis_name="s")


@functools.partial(
    pl.kernel,
    mesh=_mesh,
    out_type=jax.ShapeDtypeStruct((B_TOTAL, D), jnp.float32),
    compiler_params=pltpu.CompilerParams(use_tc_tiling_on_sc=False),
    scratch_types=[
        pltpu.VMEM((NCHUNK, CHUNK), jnp.int32),
        pltpu.VMEM((CHUNK, D), jnp.float32),
        pltpu.SemaphoreType.DMA,
    ],
)
def _emb_lookup(x_hbm, table_hbm, out_hbm, idx_v, rows_v, sem):
    wid = lax.axis_index("s") * NC + lax.axis_index("c")
    base = wid * BPW
    pltpu.sync_copy(x_hbm.at[wid], idx_v)

    def step(g, carry):
        pltpu.async_copy(table_hbm.at[idx_v.at[g]], rows_v, sem).wait()
        pltpu.sync_copy(rows_v, out_hbm.at[pl.ds(base + g * CHUNK, CHUNK)])
        return carry

    lax.fori_loop(0, NCHUNK, step, 0)


def kernel(x, table):
    xf = x.reshape(NW, NCHUNK, CHUNK).astype(jnp.int32)
    out = _emb_lookup(xf, table)
    return out.reshape(x.shape[0], x.shape[1], D)


# speedup vs baseline: 1.8652x; 1.1063x over previous
import functools

import jax
import jax.numpy as jnp
from jax import lax
from jax.experimental import pallas as pl
from jax.experimental.pallas import tpu as pltpu
from jax.experimental.pallas import tpu_sc as plsc

B_TOTAL = 16384 * 50
D = 64
NC, NS = 2, 16
NW = NC * NS
BPW = B_TOTAL // NW
GC = 512
NGRP = BPW // GC

_mesh = plsc.VectorSubcoreMesh(core_axis_name="c", subcore_axis_name="s")


@functools.partial(
    pl.kernel,
    mesh=_mesh,
    out_type=jax.ShapeDtypeStruct((NW, NGRP, GC, D), jnp.float32),
    compiler_params=pltpu.CompilerParams(use_tc_tiling_on_sc=False),
    scratch_types=[
        pltpu.VMEM((NGRP, GC), jnp.int32),
        pltpu.VMEM((2, GC, D), jnp.float32),
        pltpu.SemaphoreType.DMA,
        pltpu.SemaphoreType.DMA,
        pltpu.SemaphoreType.DMA,
        pltpu.SemaphoreType.DMA,
    ],
)
def _emb_lookup(x_hbm, table_hbm, out_hbm, idx_v, rows_v, g0, g1, w0, w1):
    wid = lax.axis_index("s") * NC + lax.axis_index("c")
    pltpu.sync_copy(x_hbm.at[wid], idx_v)

    gsem = (g0, g1)
    wsem = (w0, w1)

    def gather(t, b):
        return pltpu.make_async_copy(
            table_hbm.at[idx_v.at[t]], rows_v.at[b], gsem[b])

    def write(t, b):
        return pltpu.make_async_copy(
            rows_v.at[b], out_hbm.at[wid, t], wsem[b])

    gather(0, 0).start()

    def outer(tt, carry):
        t0 = 2 * tt
        gather(t0 + 1, 1).start()
        gather(t0, 0).wait()
        write(t0, 0).start()
        gather(t0 + 1, 1).wait()
        write(t0 + 1, 1).start()
        write(t0, 0).wait()

        @pl.when(tt < NGRP // 2 - 1)
        def _():
            gather(t0 + 2, 0).start()

        write(t0 + 1, 1).wait()
        return carry

    lax.fori_loop(0, NGRP // 2, outer, 0)


def kernel(x, table):
    xf = x.reshape(NW, NGRP, GC).astype(jnp.int32)
    out = _emb_lookup(xf, table)
    return out.reshape(x.shape[0], x.shape[1], D)


# speedup vs baseline: 1.9507x; 1.0459x over previous
import functools

import jax
import jax.numpy as jnp
from jax import lax
from jax.experimental import pallas as pl
from jax.experimental.pallas import tpu as pltpu
from jax.experimental.pallas import tpu_sc as plsc

B = 16384
S = 50
D = 64
NC, NS = 2, 16
NW = NC * NS
GC = B // NW

_mesh = plsc.VectorSubcoreMesh(core_axis_name="c", subcore_axis_name="s")


@functools.partial(
    pl.kernel,
    mesh=_mesh,
    out_type=jax.ShapeDtypeStruct((S, B, D), jnp.float32),
    compiler_params=pltpu.CompilerParams(use_tc_tiling_on_sc=False),
    scratch_types=[
        pltpu.VMEM((S, GC), jnp.int32),
        pltpu.VMEM((2, GC, D), jnp.float32),
        pltpu.SemaphoreType.DMA,
        pltpu.SemaphoreType.DMA,
        pltpu.SemaphoreType.DMA,
        pltpu.SemaphoreType.DMA,
    ],
)
def _emb_lookup(xt_hbm, table_hbm, out_hbm, idx_v, rows_v, g0, g1, w0, w1):
    wid = lax.axis_index("s") * NC + lax.axis_index("c")
    col = wid * GC
    pltpu.sync_copy(xt_hbm.at[:, pl.ds(col, GC)], idx_v)

    gsem = (g0, g1)
    wsem = (w0, w1)

    def gather(t, b):
        return pltpu.make_async_copy(
            table_hbm.at[idx_v.at[t]], rows_v.at[b], gsem[b])

    def write(t, b):
        return pltpu.make_async_copy(
            rows_v.at[b], out_hbm.at[t, pl.ds(col, GC)], wsem[b])

    gather(0, 0).start()

    def outer(tt, carry):
        t0 = 2 * tt
        gather(t0 + 1, 1).start()
        gather(t0, 0).wait()
        write(t0, 0).start()
        gather(t0 + 1, 1).wait()
        write(t0 + 1, 1).start()
        write(t0, 0).wait()

        @pl.when(tt < S // 2 - 1)
        def _():
            gather(t0 + 2, 0).start()

        write(t0 + 1, 1).wait()
        return carry

    lax.fori_loop(0, S // 2, outer, 0)


def kernel(x, table):
    out = _emb_lookup(x.T, table)
    return out.transpose(1, 0, 2)
